# baseline (speedup 1.0000x reference)
import functools

import jax
import jax.numpy as jnp
from jax import lax
from jax.experimental import pallas as pl
from jax.experimental.pallas import tpu as pltpu
from jax.experimental.pallas import tpu_sc as plsc

B = 4096
L = 200
D = 300
LANES = 16
NFULL = D // LANES
TAIL = D - NFULL * LANES
NC = 2
NS = 16
NW = NC * NS
PPW = B // NW


def _post_embedding_kernel(idx_hbm, table_hbm, out_hbm, idx_v, rows_v, out_v,
                           sem):
    wid = lax.axis_index("s") * NC + lax.axis_index("c")
    base = wid * PPW
    pltpu.sync_copy(idx_hbm.at[pl.ds(base, PPW), :], idx_v)

    iota = lax.iota(jnp.int32, LANES)
    tail_cols = jnp.minimum(NFULL * LANES + iota, D - 1)
    tail_mask = iota < TAIL

    def per_post(p, carry):
        pltpu.async_copy(table_hbm.at[idx_v.at[p]], rows_v, sem).wait()

        def per_row(s, accs):
            full = [accs[c] + rows_v[s, pl.ds(c * LANES, LANES)]
                    for c in range(NFULL)]
            srow = jnp.full((LANES,), s, dtype=jnp.int32)
            tail = plsc.load_gather(rows_v, [srow, tail_cols], mask=tail_mask)
            return tuple(full) + (accs[NFULL] + tail,)

        zero = jnp.zeros((LANES,), jnp.float32)
        accs = lax.fori_loop(0, L, per_row, (zero,) * (NFULL + 1))
        scale = jnp.float32(1.0 / L)
        for c in range(NFULL):
            out_v[p, pl.ds(c * LANES, LANES)] = accs[c] * scale
        prow = jnp.full((LANES,), p, dtype=jnp.int32)
        plsc.store_scatter(out_v, [prow, tail_cols], accs[NFULL] * scale,
                           mask=tail_mask)
        return carry

    lax.fori_loop(0, PPW, per_post, 0)
    pltpu.sync_copy(out_v, out_hbm.at[pl.ds(base, PPW), :])


@jax.jit
def _run(indices, table):
    mesh = plsc.VectorSubcoreMesh(core_axis_name="c", subcore_axis_name="s")
    kern = functools.partial(
        pl.kernel,
        mesh=mesh,
        compiler_params=pltpu.CompilerParams(use_tc_tiling_on_sc=False,
                                              needs_layout_passes=False),
        out_type=jax.ShapeDtypeStruct((B, D), jnp.float32),
        scratch_types=[
            pltpu.VMEM((PPW, L), jnp.int32),
            pltpu.VMEM((L, D), jnp.float32),
            pltpu.VMEM((PPW, D), jnp.float32),
            pltpu.SemaphoreType.DMA,
        ],
    )(_post_embedding_kernel)
    return kern(indices, table)


def kernel(indices, table):
    return _run(indices, table)


# speedup vs baseline: 1.0449x; 1.0449x over previous
import functools

import jax
import jax.numpy as jnp
from jax import lax
from jax.experimental import pallas as pl
from jax.experimental.pallas import tpu as pltpu
from jax.experimental.pallas import tpu_sc as plsc

B = 4096
L = 200
D = 300
LANES = 16
NFULL = D // LANES
TAIL = D - NFULL * LANES
NC = 2
NS = 16
NW = NC * NS
PPW = B // NW
RPT = 20
NR = L * RPT
TABROWS = 1000000 * D // LANES
LPAD = 208
OBLK = 16


def _body(idx_hbm, tabv_hbm, out_hbm, idx_v, r0_v, base_v, ridx_v, rows_v,
          out_v, sem):
    wid = lax.axis_index("s") * NC + lax.axis_index("c")
    base = wid * PPW
    pltpu.sync_copy(idx_hbm.at[pl.ds(base, PPW), :], idx_v)

    iota = lax.iota(jnp.int32, LANES)
    tsel, jcon = [], []
    for m in range(5):
        a = (16 * m) // RPT
        bound = RPT * (a + 1) - 16 * m
        t_m = jnp.int32(a) + (iota >= bound).astype(jnp.int32)
        tsel.append(t_m)
        jcon.append(jnp.int32(16 * m) + iota - jnp.int32(RPT) * t_m)
    ccol = [jnp.int32(c * LANES) + iota for c in range(NFULL + 1)]
    tail_cols = jnp.minimum(NFULL * LANES + iota, D - 1)
    tail_mask = iota < TAIL

    def per_post(p, carry):
        for c in range(LPAD // LANES):
            if c < L // LANES:
                v = idx_v[p, pl.ds(c * LANES, LANES)]
            else:
                vcols = jnp.minimum(jnp.int32(c * LANES) + iota, L - 1)
                v = plsc.load_gather(idx_v, [jnp.full((LANES,), p, jnp.int32),
                                             vcols])
            w0 = v * 300
            r0 = jnp.minimum(lax.shift_right_logical(w0, 4),
                             jnp.int32(TABROWS - RPT))
            r0_v[pl.ds(c * LANES, LANES)] = r0
            tloc = jnp.int32(c * LANES) + iota
            base_v[pl.ds(c * LANES, LANES)] = (
                tloc * (RPT * LANES) + w0 - lax.shift_left(r0, 4))

        def per_group(g, carry2):
            for m in range(5):
                t = jnp.int32(4) * g + tsel[m]
                rv = plsc.load_gather(r0_v, [t]) + jcon[m]
                ridx_v[pl.ds(80 * g + 16 * m, LANES)] = rv
            return carry2

        lax.fori_loop(0, L // 4, per_group, 0)

        pltpu.async_copy(tabv_hbm.at[ridx_v], rows_v, sem).wait()

        def per_row(t, accs):
            bsp = plsc.load_gather(base_v, [jnp.full((LANES,), t, jnp.int32)])
            new = []
            for c in range(NFULL + 1):
                w = bsp + (tail_cols if c == NFULL else ccol[c])
                new.append(accs[c] + plsc.load_gather(
                    rows_v, [lax.shift_right_logical(w, 4),
                             lax.bitwise_and(w, 15)]))
            return tuple(new)

        zero = jnp.zeros((LANES,), jnp.float32)
        accs = lax.fori_loop(0, L, per_row, (zero,) * (NFULL + 1))

        scale = jnp.float32(1.0 / L)
        prow = jnp.full((LANES,), lax.rem(p, jnp.int32(OBLK)), jnp.int32)
        for c in range(NFULL):
            plsc.store_scatter(out_v, [prow, ccol[c]], accs[c] * scale)
        plsc.store_scatter(out_v, [prow, tail_cols], accs[NFULL] * scale,
                           mask=tail_mask)

        @pl.when(lax.rem(p, jnp.int32(OBLK)) == OBLK - 1)
        def _():
            pltpu.sync_copy(out_v, out_hbm.at[pl.ds(base + p - (OBLK - 1),
                                                    OBLK), :])

        return carry

    lax.fori_loop(0, PPW, per_post, 0)


@jax.jit
def _run(indices, table):
    tabv = table.reshape(-1, LANES)
    mesh = plsc.VectorSubcoreMesh(core_axis_name="c", subcore_axis_name="s")
    kern = functools.partial(
        pl.kernel,
        mesh=mesh,
        compiler_params=pltpu.CompilerParams(use_tc_tiling_on_sc=False,
                                             needs_layout_passes=False),
        out_type=jax.ShapeDtypeStruct((B, D), jnp.float32),
        scratch_types=[
            pltpu.VMEM((PPW, L), jnp.int32),
            pltpu.VMEM((LPAD,), jnp.int32),
            pltpu.VMEM((LPAD,), jnp.int32),
            pltpu.VMEM((NR,), jnp.int32),
            pltpu.VMEM((NR, LANES), jnp.float32),
            pltpu.VMEM((OBLK, D), jnp.float32),
            pltpu.SemaphoreType.DMA,
        ],
    )(_body)
    return kern(indices, tabv)


def kernel(indices, table):
    return _run(indices, table)


# speedup vs baseline: 1.2380x; 1.1848x over previous
import functools

import jax
import jax.numpy as jnp
from jax import lax
from jax.experimental import pallas as pl
from jax.experimental.pallas import tpu as pltpu
from jax.experimental.pallas import tpu_sc as plsc

B = 4096
L = 200
D = 300
DP = 384
DO = 304
LANES = 16
NCH = DO // LANES
NC = 2
NS = 16
NW = NC * NS
PPW = B // NW
OBLK = 16


def _body(idx_hbm, tab_hbm, out_hbm, idx_v, rows_v, out_v, sem):
    wid = lax.axis_index("s") * NC + lax.axis_index("c")
    base = wid * PPW
    scale = jnp.float32(1.0 / L)
    zero = jnp.zeros((LANES,), jnp.float32)

    def per_post(p, carry):
        pltpu.sync_copy(idx_hbm.at[base + p], idx_v)
        pltpu.async_copy(tab_hbm.at[idx_v], rows_v, sem).wait()

        def per_row(t, accs):
            return tuple(accs[c] + rows_v[t, pl.ds(c * LANES, LANES)]
                         for c in range(NCH))

        accs = lax.fori_loop(0, L, per_row, (zero,) * NCH)
        pm = lax.rem(p, jnp.int32(OBLK))
        for c in range(NCH):
            out_v[pm, pl.ds(c * LANES, LANES)] = accs[c] * scale

        @pl.when(pm == OBLK - 1)
        def _():
            start = pl.multiple_of(base + p - (OBLK - 1), OBLK)
            pltpu.sync_copy(out_v, out_hbm.at[pl.ds(start, OBLK), :])

        return carry

    lax.fori_loop(0, PPW, per_post, 0)


@jax.jit
def _run(indices, table):
    tabp = jnp.pad(table, ((0, 0), (0, DP - D)))
    mesh = plsc.VectorSubcoreMesh(core_axis_name="c", subcore_axis_name="s")
    kern = functools.partial(
        pl.kernel,
        mesh=mesh,
        compiler_params=pltpu.CompilerParams(use_tc_tiling_on_sc=True,
                                             needs_layout_passes=False),
        out_type=jax.ShapeDtypeStruct((B, DO), jnp.float32),
        scratch_types=[
            pltpu.VMEM((L,), jnp.int32),
            pltpu.VMEM((L, DP), jnp.float32),
            pltpu.VMEM((OBLK, DO), jnp.float32),
            pltpu.SemaphoreType.DMA,
        ],
    )(_body)
    return kern(indices, tabp)[:, :D]


def kernel(indices, table):
    return _run(indices, table)


# speedup vs baseline: 2.6697x; 2.1565x over previous
import functools

import jax
import jax.numpy as jnp
from jax import lax
from jax.experimental import pallas as pl
from jax.experimental.pallas import tpu as pltpu
from jax.experimental.pallas import tpu_sc as plsc

B = 4096
L = 200
D = 300
DP = 384
DO = 304
LANES = 16
NCH = DO // LANES
NC = 2
NS = 16
NW = NC * NS
PPW = B // NW
OBLK = 16


def _body(idx_hbm, tab_hbm, out_hbm, idx_v, rows_v, out_v, sem):
    wid = lax.axis_index("s") * NC + lax.axis_index("c")
    base = wid * PPW
    scale = jnp.float32(1.0 / L)
    zero = jnp.zeros((LANES,), jnp.float32)

    def per_post(p, carry):
        pltpu.sync_copy(idx_hbm.at[base + p], idx_v)
        pltpu.async_copy(tab_hbm.at[idx_v], rows_v, sem).wait()

        def per_row(t, accs):
            return tuple(accs[c] + rows_v[t, pl.ds(c * LANES, LANES)]
                         for c in range(NCH))

        accs = lax.fori_loop(0, L, per_row, (zero,) * NCH)
        pm = lax.rem(p, jnp.int32(OBLK))
        for c in range(NCH):
            out_v[pm, pl.ds(c * LANES, LANES)] = accs[c] * scale

        @pl.when(pm == OBLK - 1)
        def _():
            start = pl.multiple_of(base + p - (OBLK - 1), OBLK)
            pltpu.sync_copy(out_v, out_hbm.at[pl.ds(start, OBLK), :])

        return carry

    lax.fori_loop(0, PPW, per_post, 0)


PAD_BLK = 2000


def _pad_body(t_ref, o_ref):
    o_ref[:, :D] = t_ref[...]
    o_ref[:, D:] = jnp.zeros((PAD_BLK, DP - D), jnp.float32)


def _pad_table(table):
    v = table.shape[0]
    return pl.pallas_call(
        _pad_body,
        grid=(v // PAD_BLK,),
        in_specs=[pl.BlockSpec((PAD_BLK, D), lambda i: (i, 0))],
        out_specs=pl.BlockSpec((PAD_BLK, DP), lambda i: (i, 0)),
        out_shape=jax.ShapeDtypeStruct((v, DP), jnp.float32),
    )(table)


@jax.jit
def _run(indices, table):
    tabp = _pad_table(table)
    mesh = plsc.VectorSubcoreMesh(core_axis_name="c", subcore_axis_name="s")
    kern = functools.partial(
        pl.kernel,
        mesh=mesh,
        compiler_params=pltpu.CompilerParams(use_tc_tiling_on_sc=True,
                                             needs_layout_passes=False),
        out_type=jax.ShapeDtypeStruct((B, DO), jnp.float32),
        scratch_types=[
            pltpu.VMEM((L,), jnp.int32),
            pltpu.VMEM((L, DP), jnp.float32),
            pltpu.VMEM((OBLK, DO), jnp.float32),
            pltpu.SemaphoreType.DMA,
        ],
    )(_body)
    return kern(indices, tabp)[:, :D]


def kernel(indices, table):
    return _run(indices, table)


# speedup vs baseline: 3.3802x; 1.2661x over previous
import functools

import jax
import jax.numpy as jnp
from jax import lax
from jax.experimental import pallas as pl
from jax.experimental.pallas import tpu as pltpu
from jax.experimental.pallas import tpu_sc as plsc

B = 4096
L = 200
D = 300
DP = 384
DO = 304
LANES = 16
NCH = DO // LANES
NC = 2
NS = 16
NW = NC * NS
PPW = B // NW
OBLK = 16


def _body(idx_hbm, tab_hbm, tail_hbm, out_hbm, idx_v, rows_v, out_v, sem):
    wid = lax.axis_index("s") * NC + lax.axis_index("c")
    base = wid * PPW
    scale = jnp.float32(1.0 / L)
    zero = jnp.zeros((LANES,), jnp.float32)

    def per_post(p, carry):
        pltpu.sync_copy(idx_hbm.at[base + p], idx_v)
        c0 = pltpu.async_copy(tab_hbm.at[idx_v, pl.ds(0, 128)],
                              rows_v.at[:, pl.ds(0, 128)], sem)
        c1 = pltpu.async_copy(tab_hbm.at[idx_v, pl.ds(128, 128)],
                              rows_v.at[:, pl.ds(128, 128)], sem)
        c2 = pltpu.async_copy(tail_hbm.at[idx_v],
                              rows_v.at[:, pl.ds(256, 128)], sem)
        c0.wait()
        c1.wait()
        c2.wait()

        def per_row(t, accs):
            return tuple(accs[c] + rows_v[t, pl.ds(c * LANES, LANES)]
                         for c in range(NCH))

        accs = lax.fori_loop(0, L, per_row, (zero,) * NCH)
        pm = lax.rem(p, jnp.int32(OBLK))
        for c in range(NCH):
            out_v[pm, pl.ds(c * LANES, LANES)] = accs[c] * scale

        @pl.when(pm == OBLK - 1)
        def _():
            start = pl.multiple_of(base + p - (OBLK - 1), OBLK)
            pltpu.sync_copy(out_v, out_hbm.at[pl.ds(start, OBLK), :])

        return carry

    lax.fori_loop(0, PPW, per_post, 0)


PAD_BLK = 8000


def _tail_body(t_ref, o_ref):
    o_ref[...] = t_ref[...]


def _tail_table(table):
    v = table.shape[0]
    return pl.pallas_call(
        _tail_body,
        grid=(v // PAD_BLK,),
        in_specs=[pl.BlockSpec((PAD_BLK, 128), lambda i: (i, 2))],
        out_specs=pl.BlockSpec((PAD_BLK, 128), lambda i: (i, 0)),
        out_shape=jax.ShapeDtypeStruct((v, 128), jnp.float32),
    )(table)


@jax.jit
def _run(indices, table):
    tail = _tail_table(table)
    mesh = plsc.VectorSubcoreMesh(core_axis_name="c", subcore_axis_name="s")
    kern = functools.partial(
        pl.kernel,
        mesh=mesh,
        compiler_params=pltpu.CompilerParams(use_tc_tiling_on_sc=True,
                                             needs_layout_passes=False),
        out_type=jax.ShapeDtypeStruct((B, DO), jnp.float32),
        scratch_types=[
            pltpu.VMEM((L,), jnp.int32),
            pltpu.VMEM((L, DP), jnp.float32),
            pltpu.VMEM((OBLK, DO), jnp.float32),
            pltpu.SemaphoreType.DMA,
        ],
    )(_body)
    return kern(indices, table, tail)[:, :D]


def kernel(indices, table):
    return _run(indices, table)
